# Initial kernel scaffold; baseline (speedup 1.0000x reference)
import jax
import jax.numpy as jnp
from jax.experimental import pallas as pl


def kernel(x, edge_index, batch, global_feats, W0, b0, g0, be0, W1, b1, g1, be1, W2, b2, g2, be2, mW0, mb0, mW1, mb1, mW2, mb2):
    raise NotImplementedError("write your pallas kernel here")


# speedup vs baseline: 8.1111x; 8.1111x over previous
import functools

import jax
import jax.numpy as jnp
from jax import lax
from jax.experimental import pallas as pl
from jax.experimental.pallas import tpu as pltpu
from jax.experimental.pallas import tpu_sc as plsc

N = 10000
D = 128
HD = D // 2
B = 64
NC = 2
NS = 16
NW = NC * NS
CHUNK = 128
N_ACC = 10112
ZROWS = N_ACC // NS

_HI = jax.lax.Precision.HIGHEST


@functools.cache
def _mesh():
    return plsc.VectorSubcoreMesh(core_axis_name="c", subcore_axis_name="s",
                                  num_cores=NC, num_subcores=NS)


def _sc_aggregate_body(u_hbm, src_hbm, dst_hbm, zeros_hbm, out_hbm,
                       src_v, dst_v, buf0, buf1, acc, sem0, sem1):
    c = lax.axis_index("c")
    s = lax.axis_index("s")
    wid = s * NC + c
    cpt = src_v.shape[0]
    pltpu.sync_copy(src_hbm.at[wid], src_v)
    pltpu.sync_copy(dst_hbm.at[wid], dst_v)

    for k in range(2):
        uk = u_hbm.at[k]
        pltpu.sync_copy(zeros_hbm, acc.at[pl.ds(s * ZROWS, ZROWS)])
        plsc.subcore_barrier()

        pltpu.async_copy(uk.at[src_v.at[0]], buf0, sem0)

        def body(j, carry):
            @pl.when(j + 1 < cpt)
            def _():
                @pl.when(lax.rem(j, 2) == 0)
                def _():
                    pltpu.async_copy(uk.at[src_v.at[j + 1]], buf1, sem1)

                @pl.when(lax.rem(j, 2) == 1)
                def _():
                    pltpu.async_copy(uk.at[src_v.at[j + 1]], buf0, sem0)

            @pl.when(lax.rem(j, 2) == 0)
            def _():
                pltpu.make_async_copy(uk.at[src_v.at[0]], buf0, sem0).wait()
                pltpu.sync_copy(buf0, acc.at[dst_v.at[j]], add=True)

            @pl.when(lax.rem(j, 2) == 1)
            def _():
                pltpu.make_async_copy(uk.at[src_v.at[0]], buf1, sem1).wait()
                pltpu.sync_copy(buf1, acc.at[dst_v.at[j]], add=True)

            return carry

        lax.fori_loop(0, cpt, body, 0)
        plsc.subcore_barrier()
        pltpu.sync_copy(acc.at[pl.ds(s * ZROWS, ZROWS)],
                        out_hbm.at[c, k, pl.ds(s * ZROWS, ZROWS)])


def _make_sc_aggregate(cpt):
    return functools.partial(
        pl.kernel, _sc_aggregate_body,
        out_type=jax.ShapeDtypeStruct((NC, 2, N_ACC, HD), jnp.float32),
        mesh=_mesh(),
        scratch_types=[
            pltpu.VMEM((cpt, CHUNK), jnp.int32),
            pltpu.VMEM((cpt, CHUNK), jnp.int32),
            pltpu.VMEM((CHUNK, HD), jnp.float32),
            pltpu.VMEM((CHUNK, HD), jnp.float32),
            pltpu.VMEM_SHARED((N_ACC, HD), jnp.float32),
            pltpu.SemaphoreType.DMA,
            pltpu.SemaphoreType.DMA,
        ],
        compiler_params=pltpu.CompilerParams(use_tc_tiling_on_sc=False),
    )()


RCH = 2000
NBLK = N // RCH


def _tc_prep_body(accp_ref, x_ref, dinv_ref, u_ref):
    deg = accp_ref[0, 0, :, 0:1] + accp_ref[1, 0, :, 0:1] + 1.0
    dinv = jnp.where(deg > 0, 1.0 / jnp.sqrt(deg), 0.0)
    dinv_ref[...] = dinv
    x = x_ref[...]
    u_ref[0] = x[:, 0:HD] * dinv
    u_ref[1] = x[:, HD:D] * dinv


_tc_prep = pl.pallas_call(
    _tc_prep_body,
    grid=(NBLK,),
    in_specs=[
        pl.BlockSpec((2, 2, RCH, HD), lambda i: (0, 0, i, 0)),
        pl.BlockSpec((RCH, D), lambda i: (i, 0)),
    ],
    out_specs=[
        pl.BlockSpec((RCH, 1), lambda i: (i, 0)),
        pl.BlockSpec((2, RCH, HD), lambda i: (0, i, 0)),
    ],
    out_shape=[jax.ShapeDtypeStruct((N, 1), jnp.float32),
               jax.ShapeDtypeStruct((2, N, HD), jnp.float32)],
)


def _zblock(accp_ref, u_ref, dinv_ref, w_ref, b_ref):
    dinv = dinv_ref[...]
    t0 = (accp_ref[0, 0] + accp_ref[1, 0] + u_ref[0]) * dinv
    t1 = (accp_ref[0, 1] + accp_ref[1, 1] + u_ref[1]) * dinv
    z = jnp.dot(t0, w_ref[0:HD, :], preferred_element_type=jnp.float32,
                precision=_HI)
    z = z + jnp.dot(t1, w_ref[HD:D, :], preferred_element_type=jnp.float32,
                    precision=_HI)
    return z + b_ref[...]


def _tc_stats_body(accp_ref, u_ref, dinv_ref, w_ref, b_ref, s1_ref, s2_ref):
    i = pl.program_id(0)
    z = _zblock(accp_ref, u_ref, dinv_ref, w_ref, b_ref)

    @pl.when(i == 0)
    def _():
        s1_ref[...] = jnp.zeros_like(s1_ref)
        s2_ref[...] = jnp.zeros_like(s2_ref)

    s1_ref[...] += jnp.sum(z, axis=0, keepdims=True)
    s2_ref[...] += jnp.sum(z * z, axis=0, keepdims=True)


_tc_stats = pl.pallas_call(
    _tc_stats_body,
    grid=(NBLK,),
    in_specs=[
        pl.BlockSpec((2, 2, RCH, HD), lambda i: (0, 0, i, 0)),
        pl.BlockSpec((2, RCH, HD), lambda i: (0, i, 0)),
        pl.BlockSpec((RCH, 1), lambda i: (i, 0)),
        pl.BlockSpec((D, D), lambda i: (0, 0)),
        pl.BlockSpec((1, D), lambda i: (0, 0)),
    ],
    out_specs=[
        pl.BlockSpec((1, D), lambda i: (0, 0)),
        pl.BlockSpec((1, D), lambda i: (0, 0)),
    ],
    out_shape=[jax.ShapeDtypeStruct((1, D), jnp.float32),
               jax.ShapeDtypeStruct((1, D), jnp.float32)],
)


def _tc_norm_body(accp_ref, u_ref, dinv_ref, w_ref, b_ref, g_ref, be_ref,
                  s1_ref, s2_ref, h_ref, un_ref):
    z = _zblock(accp_ref, u_ref, dinv_ref, w_ref, b_ref)
    m = s1_ref[...] / N
    var = s2_ref[...] / N - m * m
    rstd = 1.0 / jnp.sqrt(var + 1e-5)
    h = jnp.maximum((z - m) * rstd * g_ref[...] + be_ref[...], 0.0)
    h_ref[...] = h
    hd = h * dinv_ref[...]
    un_ref[0] = hd[:, 0:HD]
    un_ref[1] = hd[:, HD:D]


_tc_norm = pl.pallas_call(
    _tc_norm_body,
    grid=(NBLK,),
    in_specs=[
        pl.BlockSpec((2, 2, RCH, HD), lambda i: (0, 0, i, 0)),
        pl.BlockSpec((2, RCH, HD), lambda i: (0, i, 0)),
        pl.BlockSpec((RCH, 1), lambda i: (i, 0)),
        pl.BlockSpec((D, D), lambda i: (0, 0)),
        pl.BlockSpec((1, D), lambda i: (0, 0)),
        pl.BlockSpec((1, D), lambda i: (0, 0)),
        pl.BlockSpec((1, D), lambda i: (0, 0)),
        pl.BlockSpec((1, D), lambda i: (0, 0)),
        pl.BlockSpec((1, D), lambda i: (0, 0)),
    ],
    out_specs=[
        pl.BlockSpec((RCH, D), lambda i: (i, 0)),
        pl.BlockSpec((2, RCH, HD), lambda i: (0, i, 0)),
    ],
    out_shape=[jax.ShapeDtypeStruct((N, D), jnp.float32),
               jax.ShapeDtypeStruct((2, N, HD), jnp.float32)],
)


def _tc_layer(accp, ui, dinv, W, b, g, be):
    s1, s2 = _tc_stats(accp, ui, dinv, W, b)
    return _tc_norm(accp, ui, dinv, W, b, g, be, s1, s2)


def _tc_head_body(h_ref, batch_ref, gf_ref, w0_ref, b0_ref, w1_ref, b1_ref,
                  w2t_ref, b2_ref, out_ref):
    rows = lax.broadcasted_iota(jnp.int32, (B, N), 0)
    oh = (batch_ref[...] == rows).astype(jnp.float32)
    sums = jnp.dot(oh, h_ref[...], preferred_element_type=jnp.float32,
                   precision=_HI)
    cnt = jnp.sum(oh, axis=1, keepdims=True)
    pooled = sums / jnp.maximum(cnt, 1.0)
    z = jnp.dot(pooled, w0_ref[0:D, :], preferred_element_type=jnp.float32,
                precision=_HI)
    z = z + jnp.dot(gf_ref[...], w0_ref[D:, :],
                    preferred_element_type=jnp.float32, precision=_HI)
    z = jnp.maximum(z + b0_ref[...], 0.0)
    z = jnp.maximum(jnp.dot(z, w1_ref[...], preferred_element_type=jnp.float32,
                            precision=_HI) + b1_ref[...], 0.0)
    out_ref[...] = jnp.sum(z * w2t_ref[...], axis=1, keepdims=True) + b2_ref[...]


_tc_head = pl.pallas_call(
    _tc_head_body,
    out_shape=jax.ShapeDtypeStruct((B, 1), jnp.float32),
)


@jax.jit
def kernel(x, edge_index, batch, global_feats,
           W0, b0, g0, be0, W1, b1, g1, be1, W2, b2, g2, be2,
           mW0, mb0, mW1, mb1, mW2, mb2):
    src = edge_index[0]
    dst = edge_index[1]
    e = src.shape[0]
    cpt = -(-e // (NW * CHUNK))
    e_pad = NW * cpt * CHUNK
    pad = e_pad - e
    src_p = jnp.concatenate(
        [src, jnp.zeros((pad,), jnp.int32)]).reshape(NW, cpt, CHUNK)
    dst_p = jnp.concatenate(
        [dst, jnp.full((pad,), N, jnp.int32)]).reshape(NW, cpt, CHUNK)

    zeros_h = jnp.zeros((ZROWS, HD), jnp.float32)

    agg = _make_sc_aggregate(cpt)
    Ws = jnp.stack([W0, W0, W1, W2])
    bs = jnp.stack([b0, b0, b1, b2]).reshape(4, 1, -1)
    gs = jnp.stack([g0, g0, g1, g2]).reshape(4, 1, -1)
    bes = jnp.stack([be0, be0, be1, be2]).reshape(4, 1, -1)
    idx = jnp.arange(4, dtype=jnp.int32)

    def step(carry, wgts):
        ui, hi, dinvi = carry
        i, W, b, g, be = wgts
        accp = agg(ui, src_p, dst_p, zeros_h)

        def first(_):
            dinv, u0 = _tc_prep(accp, x)
            return (u0, hi, dinv)

        def rest(_):
            hn, un = _tc_layer(accp, ui, dinvi, W, b, g, be)
            return (un, hn, dinvi)

        return lax.cond(i == 0, first, rest, None), None

    carry0 = (jnp.ones((2, N, HD), jnp.float32),
              jnp.zeros((N, D), jnp.float32),
              jnp.zeros((N, 1), jnp.float32))
    (_, h, _), _ = lax.scan(step, carry0, (idx, Ws, bs, gs, bes))

    out = _tc_head(h, batch.reshape(1, N), global_feats,
                   mW0, mb0.reshape(1, -1), mW1, mb1.reshape(1, -1),
                   mW2.reshape(1, -1), mb2.reshape(1, 1))
    return out.reshape(-1)


# speedup vs baseline: 17.6145x; 2.1716x over previous
import functools

import jax
import jax.numpy as jnp
from jax import lax
from jax.experimental import pallas as pl
from jax.experimental.pallas import tpu as pltpu
from jax.experimental.pallas import tpu_sc as plsc

N = 10000
D = 128
HD = D // 2
B = 64
NC = 2
NS = 16
NW = NC * NS
CHUNK = 128
N_ACC = 10112
ZROWS = N_ACC // NS

_HI = jax.lax.Precision.HIGHEST


@functools.cache
def _mesh():
    return plsc.VectorSubcoreMesh(core_axis_name="c", subcore_axis_name="s",
                                  num_cores=NC, num_subcores=NS)


NBUF = 4


def _sc_aggregate_body(mode_hbm, u_hbm, src_hbm, dst_hbm, zeros_hbm, ones_hbm,
                       out_hbm, mode_v, src_v, dst_v, ones_v, bufs, acc,
                       gsems, ssems):
    c = lax.axis_index("c")
    s = lax.axis_index("s")
    wid = s * NC + c
    cpt = src_v.shape[0]
    pltpu.sync_copy(mode_hbm, mode_v)
    pltpu.sync_copy(src_hbm.at[wid], src_v)
    pltpu.sync_copy(dst_hbm.at[wid], dst_v)
    deg_mode = mode_v[...][0] == 1

    @pl.when(deg_mode)
    def _():
        pltpu.sync_copy(ones_hbm, ones_v)
        pltpu.sync_copy(zeros_hbm, acc.at[pl.ds(s * ZROWS, ZROWS)])
        plsc.subcore_barrier()

        def dbody(j, carry):
            pltpu.sync_copy(ones_v, acc.at[dst_v.at[j]], add=True)
            return carry

        lax.fori_loop(0, cpt, dbody, 0)
        plsc.subcore_barrier()
        pltpu.sync_copy(acc.at[pl.ds(s * ZROWS, ZROWS)],
                        out_hbm.at[c, 0, pl.ds(s * ZROWS, ZROWS)])

    @pl.when(jnp.logical_not(deg_mode))
    def _():
        for k in range(2):
            uk = u_hbm.at[k]
            pltpu.sync_copy(zeros_hbm, acc.at[pl.ds(s * ZROWS, ZROWS)])
            plsc.subcore_barrier()

            for b in range(NBUF - 1):
                pltpu.async_copy(uk.at[src_v.at[b]], bufs[b], gsems[b])

            def body(j, carry):
                jm = lax.rem(j, NBUF)
                for b in range(NBUF):
                    @pl.when(jm == b)
                    def _(b=b):
                        pltpu.make_async_copy(
                            uk.at[src_v.at[0]], bufs[b], gsems[b]).wait()
                        pltpu.async_copy(bufs[b], acc.at[dst_v.at[j]],
                                         ssems[b], add=True)

                        @pl.when(j + NBUF - 1 < cpt)
                        def _():
                            nb = (b + NBUF - 1) % NBUF

                            @pl.when(j >= 1)
                            def _():
                                pltpu.make_async_copy(
                                    bufs[nb], acc.at[dst_v.at[0]],
                                    ssems[nb]).wait()

                            pltpu.async_copy(uk.at[src_v.at[j + NBUF - 1]],
                                             bufs[nb], gsems[nb])
                return carry

            lax.fori_loop(0, cpt, body, 0)
            for b in range(NBUF):
                pltpu.make_async_copy(bufs[b], acc.at[dst_v.at[0]],
                                      ssems[b]).wait()
            plsc.subcore_barrier()
            pltpu.sync_copy(acc.at[pl.ds(s * ZROWS, ZROWS)],
                            out_hbm.at[c, k, pl.ds(s * ZROWS, ZROWS)])


def _make_sc_aggregate(cpt):
    return functools.partial(
        pl.kernel, _sc_aggregate_body,
        out_type=jax.ShapeDtypeStruct((NC, 2, N_ACC, HD), jnp.float32),
        mesh=_mesh(),
        scratch_types=[
            pltpu.VMEM((16,), jnp.int32),
            pltpu.VMEM((cpt, CHUNK), jnp.int32),
            pltpu.VMEM((cpt, CHUNK), jnp.int32),
            pltpu.VMEM((CHUNK, HD), jnp.float32),
            [pltpu.VMEM((CHUNK, HD), jnp.float32) for _ in range(NBUF)],
            pltpu.VMEM_SHARED((N_ACC, HD), jnp.float32),
            [pltpu.SemaphoreType.DMA for _ in range(NBUF)],
            [pltpu.SemaphoreType.DMA for _ in range(NBUF)],
        ],
        compiler_params=pltpu.CompilerParams(use_tc_tiling_on_sc=False),
    )()


RCH = 2000
NBLK = N // RCH


def _tc_prep_body(accp_ref, x_ref, dinv_ref, u_ref):
    deg = accp_ref[0, 0, :, 0:1] + accp_ref[1, 0, :, 0:1] + 1.0
    dinv = jnp.where(deg > 0, 1.0 / jnp.sqrt(deg), 0.0)
    dinv_ref[...] = dinv
    x = x_ref[...]
    u_ref[0] = x[:, 0:HD] * dinv
    u_ref[1] = x[:, HD:D] * dinv


_tc_prep = pl.pallas_call(
    _tc_prep_body,
    grid=(NBLK,),
    in_specs=[
        pl.BlockSpec((2, 2, RCH, HD), lambda i: (0, 0, i, 0)),
        pl.BlockSpec((RCH, D), lambda i: (i, 0)),
    ],
    out_specs=[
        pl.BlockSpec((RCH, 1), lambda i: (i, 0)),
        pl.BlockSpec((2, RCH, HD), lambda i: (0, i, 0)),
    ],
    out_shape=[jax.ShapeDtypeStruct((N, 1), jnp.float32),
               jax.ShapeDtypeStruct((2, N, HD), jnp.float32)],
)


def _zblock(accp_ref, u_ref, dinv_ref, w_ref, b_ref):
    dinv = dinv_ref[...]
    t0 = (accp_ref[0, 0] + accp_ref[1, 0] + u_ref[0]) * dinv
    t1 = (accp_ref[0, 1] + accp_ref[1, 1] + u_ref[1]) * dinv
    z = jnp.dot(t0, w_ref[0:HD, :], preferred_element_type=jnp.float32,
                precision=_HI)
    z = z + jnp.dot(t1, w_ref[HD:D, :], preferred_element_type=jnp.float32,
                    precision=_HI)
    return z + b_ref[...]


def _tc_stats_body(accp_ref, u_ref, dinv_ref, w_ref, b_ref, s1_ref, s2_ref):
    i = pl.program_id(0)
    z = _zblock(accp_ref, u_ref, dinv_ref, w_ref, b_ref)

    @pl.when(i == 0)
    def _():
        s1_ref[...] = jnp.zeros_like(s1_ref)
        s2_ref[...] = jnp.zeros_like(s2_ref)

    s1_ref[...] += jnp.sum(z, axis=0, keepdims=True)
    s2_ref[...] += jnp.sum(z * z, axis=0, keepdims=True)


_tc_stats = pl.pallas_call(
    _tc_stats_body,
    grid=(NBLK,),
    in_specs=[
        pl.BlockSpec((2, 2, RCH, HD), lambda i: (0, 0, i, 0)),
        pl.BlockSpec((2, RCH, HD), lambda i: (0, i, 0)),
        pl.BlockSpec((RCH, 1), lambda i: (i, 0)),
        pl.BlockSpec((D, D), lambda i: (0, 0)),
        pl.BlockSpec((1, D), lambda i: (0, 0)),
    ],
    out_specs=[
        pl.BlockSpec((1, D), lambda i: (0, 0)),
        pl.BlockSpec((1, D), lambda i: (0, 0)),
    ],
    out_shape=[jax.ShapeDtypeStruct((1, D), jnp.float32),
               jax.ShapeDtypeStruct((1, D), jnp.float32)],
)


def _tc_norm_body(accp_ref, u_ref, dinv_ref, w_ref, b_ref, g_ref, be_ref,
                  s1_ref, s2_ref, h_ref, un_ref):
    z = _zblock(accp_ref, u_ref, dinv_ref, w_ref, b_ref)
    m = s1_ref[...] / N
    var = s2_ref[...] / N - m * m
    rstd = 1.0 / jnp.sqrt(var + 1e-5)
    h = jnp.maximum((z - m) * rstd * g_ref[...] + be_ref[...], 0.0)
    h_ref[...] = h
    hd = h * dinv_ref[...]
    un_ref[0] = hd[:, 0:HD]
    un_ref[1] = hd[:, HD:D]


_tc_norm = pl.pallas_call(
    _tc_norm_body,
    grid=(NBLK,),
    in_specs=[
        pl.BlockSpec((2, 2, RCH, HD), lambda i: (0, 0, i, 0)),
        pl.BlockSpec((2, RCH, HD), lambda i: (0, i, 0)),
        pl.BlockSpec((RCH, 1), lambda i: (i, 0)),
        pl.BlockSpec((D, D), lambda i: (0, 0)),
        pl.BlockSpec((1, D), lambda i: (0, 0)),
        pl.BlockSpec((1, D), lambda i: (0, 0)),
        pl.BlockSpec((1, D), lambda i: (0, 0)),
        pl.BlockSpec((1, D), lambda i: (0, 0)),
        pl.BlockSpec((1, D), lambda i: (0, 0)),
    ],
    out_specs=[
        pl.BlockSpec((RCH, D), lambda i: (i, 0)),
        pl.BlockSpec((2, RCH, HD), lambda i: (0, i, 0)),
    ],
    out_shape=[jax.ShapeDtypeStruct((N, D), jnp.float32),
               jax.ShapeDtypeStruct((2, N, HD), jnp.float32)],
)


def _tc_layer(accp, ui, dinv, W, b, g, be):
    s1, s2 = _tc_stats(accp, ui, dinv, W, b)
    return _tc_norm(accp, ui, dinv, W, b, g, be, s1, s2)


def _tc_head_body(h_ref, batch_ref, gf_ref, w0_ref, b0_ref, w1_ref, b1_ref,
                  w2t_ref, b2_ref, out_ref):
    rows = lax.broadcasted_iota(jnp.int32, (B, N), 0)
    oh = (batch_ref[...] == rows).astype(jnp.float32)
    sums = jnp.dot(oh, h_ref[...], preferred_element_type=jnp.float32,
                   precision=_HI)
    cnt = jnp.sum(oh, axis=1, keepdims=True)
    pooled = sums / jnp.maximum(cnt, 1.0)
    z = jnp.dot(pooled, w0_ref[0:D, :], preferred_element_type=jnp.float32,
                precision=_HI)
    z = z + jnp.dot(gf_ref[...], w0_ref[D:, :],
                    preferred_element_type=jnp.float32, precision=_HI)
    z = jnp.maximum(z + b0_ref[...], 0.0)
    z = jnp.maximum(jnp.dot(z, w1_ref[...], preferred_element_type=jnp.float32,
                            precision=_HI) + b1_ref[...], 0.0)
    out_ref[...] = jnp.sum(z * w2t_ref[...], axis=1, keepdims=True) + b2_ref[...]


_tc_head = pl.pallas_call(
    _tc_head_body,
    out_shape=jax.ShapeDtypeStruct((B, 1), jnp.float32),
)


@jax.jit
def kernel(x, edge_index, batch, global_feats,
           W0, b0, g0, be0, W1, b1, g1, be1, W2, b2, g2, be2,
           mW0, mb0, mW1, mb1, mW2, mb2):
    src = edge_index[0]
    dst = edge_index[1]
    e = src.shape[0]
    cpt = -(-e // (NW * CHUNK))
    cpt = -(-cpt // NBUF) * NBUF
    e_pad = NW * cpt * CHUNK
    pad = e_pad - e
    pad_ar = jnp.arange(pad, dtype=jnp.int32)
    src_p = jnp.concatenate(
        [src, pad_ar % N]).reshape(NW, cpt, CHUNK)
    dst_p = jnp.concatenate(
        [dst, N + pad_ar % (N_ACC - N)]).reshape(NW, cpt, CHUNK)

    zeros_h = jnp.zeros((ZROWS, HD), jnp.float32)
    ones_h = jnp.ones((CHUNK, HD), jnp.float32)

    agg = _make_sc_aggregate(cpt)
    Ws = jnp.stack([W0, W0, W1, W2])
    bs = jnp.stack([b0, b0, b1, b2]).reshape(4, 1, -1)
    gs = jnp.stack([g0, g0, g1, g2]).reshape(4, 1, -1)
    bes = jnp.stack([be0, be0, be1, be2]).reshape(4, 1, -1)
    idx = jnp.arange(4, dtype=jnp.int32)
    modes = jnp.concatenate([jnp.ones((1, 16), jnp.int32),
                             jnp.zeros((3, 16), jnp.int32)])

    def step(carry, wgts):
        ui, hi, dinvi = carry
        i, mode, W, b, g, be = wgts
        accp = agg(mode, ui, src_p, dst_p, zeros_h, ones_h)

        def first(_):
            dinv, u0 = _tc_prep(accp, x)
            return (u0, hi, dinv)

        def rest(_):
            hn, un = _tc_layer(accp, ui, dinvi, W, b, g, be)
            return (un, hn, dinvi)

        return lax.cond(i == 0, first, rest, None), None

    carry0 = (jnp.ones((2, N, HD), jnp.float32),
              jnp.zeros((N, D), jnp.float32),
              jnp.zeros((N, 1), jnp.float32))
    (_, h, _), _ = lax.scan(step, carry0, (idx, modes, Ws, bs, gs, bes))

    out = _tc_head(h, batch.reshape(1, N), global_feats,
                   mW0, mb0.reshape(1, -1), mW1, mb1.reshape(1, -1),
                   mW2.reshape(1, -1), mb2.reshape(1, 1))
    return out.reshape(-1)


# speedup vs baseline: 17.6755x; 1.0035x over previous
import functools

import jax
import jax.numpy as jnp
from jax import lax
from jax.experimental import pallas as pl
from jax.experimental.pallas import tpu as pltpu
from jax.experimental.pallas import tpu_sc as plsc

N = 10000
D = 128
HD = D // 2
B = 64
NC = 2
NS = 16
NW = NC * NS
CHUNK = 128
N_ACC = 10112
ZROWS = N_ACC // NS

_HI = jax.lax.Precision.HIGHEST


@functools.cache
def _mesh():
    return plsc.VectorSubcoreMesh(core_axis_name="c", subcore_axis_name="s",
                                  num_cores=NC, num_subcores=NS)


NBUF = 4


def _sc_aggregate_body(mode_hbm, u_hbm, src_hbm, dst_hbm, zeros_hbm, ones_hbm,
                       out_hbm, mode_v, src_v, dst_v, ones_v, bufs, acc,
                       gsems, ssems):
    c = lax.axis_index("c")
    s = lax.axis_index("s")
    wid = s * NC + c
    cpt = src_v.shape[0]
    pltpu.sync_copy(mode_hbm, mode_v)
    pltpu.sync_copy(src_hbm.at[wid], src_v)
    pltpu.sync_copy(dst_hbm.at[wid], dst_v)
    deg_mode = mode_v[...][0] == 1

    @pl.when(deg_mode)
    def _():
        pltpu.sync_copy(ones_hbm, ones_v)
        pltpu.sync_copy(zeros_hbm, acc.at[pl.ds(s * ZROWS, ZROWS)])
        plsc.subcore_barrier()

        def dbody(j, carry):
            pltpu.sync_copy(ones_v, acc.at[dst_v.at[j]], add=True)
            return carry

        lax.fori_loop(0, cpt, dbody, 0)
        plsc.subcore_barrier()
        pltpu.sync_copy(acc.at[pl.ds(s * ZROWS, ZROWS)],
                        out_hbm.at[c, 0, pl.ds(s * ZROWS, ZROWS)])

    @pl.when(jnp.logical_not(deg_mode))
    def _():
        for k in range(2):
            uk = u_hbm.at[k]
            pltpu.sync_copy(zeros_hbm, acc.at[pl.ds(s * ZROWS, ZROWS)])
            plsc.subcore_barrier()

            for b in range(NBUF - 1):
                pltpu.async_copy(uk.at[src_v.at[b]], bufs[b], gsems[b])

            def body(j, carry):
                jm = lax.rem(j, NBUF)
                for b in range(NBUF):
                    @pl.when(jm == b)
                    def _(b=b):
                        pltpu.make_async_copy(
                            uk.at[src_v.at[0]], bufs[b], gsems[b]).wait()
                        pltpu.async_copy(bufs[b], acc.at[dst_v.at[j]],
                                         ssems[b], add=True)

                        @pl.when(j + NBUF - 1 < cpt)
                        def _():
                            nb = (b + NBUF - 1) % NBUF

                            @pl.when(j >= 1)
                            def _():
                                pltpu.make_async_copy(
                                    bufs[nb], acc.at[dst_v.at[0]],
                                    ssems[nb]).wait()

                            pltpu.async_copy(uk.at[src_v.at[j + NBUF - 1]],
                                             bufs[nb], gsems[nb])
                return carry

            lax.fori_loop(0, cpt, body, 0)
            for b in range(NBUF):
                pltpu.make_async_copy(bufs[b], acc.at[dst_v.at[0]],
                                      ssems[b]).wait()
            plsc.subcore_barrier()
            pltpu.sync_copy(acc.at[pl.ds(s * ZROWS, ZROWS)],
                            out_hbm.at[c, k, pl.ds(s * ZROWS, ZROWS)])


def _make_sc_aggregate(cpt):
    return functools.partial(
        pl.kernel, _sc_aggregate_body,
        out_type=jax.ShapeDtypeStruct((NC, 2, N_ACC, HD), jnp.float32),
        mesh=_mesh(),
        scratch_types=[
            pltpu.VMEM((16,), jnp.int32),
            pltpu.VMEM((cpt, CHUNK), jnp.int32),
            pltpu.VMEM((cpt, CHUNK), jnp.int32),
            pltpu.VMEM((CHUNK, HD), jnp.float32),
            [pltpu.VMEM((CHUNK, HD), jnp.float32) for _ in range(NBUF)],
            pltpu.VMEM_SHARED((N_ACC, HD), jnp.float32),
            [pltpu.SemaphoreType.DMA for _ in range(NBUF)],
            [pltpu.SemaphoreType.DMA for _ in range(NBUF)],
        ],
        compiler_params=pltpu.CompilerParams(use_tc_tiling_on_sc=False),
    )()


RCH = 2000
NBLK = N // RCH


def _tc_prep_body(accp_ref, x_ref, dinv_ref, u_ref):
    deg = accp_ref[0, 0, :, 0:1] + accp_ref[1, 0, :, 0:1] + 1.0
    dinv = jnp.where(deg > 0, 1.0 / jnp.sqrt(deg), 0.0)
    dinv_ref[...] = dinv
    x = x_ref[...]
    u_ref[0] = x[:, 0:HD] * dinv
    u_ref[1] = x[:, HD:D] * dinv


_tc_prep = pl.pallas_call(
    _tc_prep_body,
    grid=(NBLK,),
    in_specs=[
        pl.BlockSpec((2, 2, RCH, HD), lambda i: (0, 0, i, 0)),
        pl.BlockSpec((RCH, D), lambda i: (i, 0)),
    ],
    out_specs=[
        pl.BlockSpec((RCH, 1), lambda i: (i, 0)),
        pl.BlockSpec((2, RCH, HD), lambda i: (0, i, 0)),
    ],
    out_shape=[jax.ShapeDtypeStruct((N, 1), jnp.float32),
               jax.ShapeDtypeStruct((2, N, HD), jnp.float32)],
)


def _zblock(accp_ref, u_ref, dinv_ref, w_ref, b_ref):
    dinv = dinv_ref[...]
    t0 = (accp_ref[0, 0] + accp_ref[1, 0] + u_ref[0]) * dinv
    t1 = (accp_ref[0, 1] + accp_ref[1, 1] + u_ref[1]) * dinv
    z = jnp.dot(t0, w_ref[0:HD, :], preferred_element_type=jnp.float32,
                precision=_HI)
    z = z + jnp.dot(t1, w_ref[HD:D, :], preferred_element_type=jnp.float32,
                    precision=_HI)
    return z + b_ref[...]


def _tc_layer_body(accp_ref, u_ref, dinv_ref, w_ref, b_ref, g_ref, be_ref,
                   h_ref, un_ref, s1_ref, s2_ref):
    i = pl.program_id(0)

    @pl.when(i == 0)
    def _():
        s1_ref[...] = jnp.zeros_like(s1_ref)
        s2_ref[...] = jnp.zeros_like(s2_ref)

    z = _zblock(accp_ref, u_ref, dinv_ref, w_ref, b_ref)

    @pl.when(i < NBLK)
    def _():
        s1_ref[...] += jnp.sum(z, axis=0, keepdims=True)
        s2_ref[...] += jnp.sum(z * z, axis=0, keepdims=True)

    @pl.when(i >= NBLK)
    def _():
        m = s1_ref[...] / N
        var = s2_ref[...] / N - m * m
        rstd = 1.0 / jnp.sqrt(var + 1e-5)
        h = jnp.maximum((z - m) * rstd * g_ref[...] + be_ref[...], 0.0)
        h_ref[...] = h
        hd = h * dinv_ref[...]
        un_ref[0] = hd[:, 0:HD]
        un_ref[1] = hd[:, HD:D]


def _blk(i):
    return lax.rem(i, NBLK)


def _oblk(i):
    return jnp.maximum(i - NBLK, 0)


_tc_layer = pl.pallas_call(
    _tc_layer_body,
    grid=(2 * NBLK,),
    in_specs=[
        pl.BlockSpec((2, 2, RCH, HD), lambda i: (0, 0, _blk(i), 0)),
        pl.BlockSpec((2, RCH, HD), lambda i: (0, _blk(i), 0)),
        pl.BlockSpec((RCH, 1), lambda i: (_blk(i), 0)),
        pl.BlockSpec((D, D), lambda i: (0, 0)),
        pl.BlockSpec((1, D), lambda i: (0, 0)),
        pl.BlockSpec((1, D), lambda i: (0, 0)),
        pl.BlockSpec((1, D), lambda i: (0, 0)),
    ],
    out_specs=[
        pl.BlockSpec((RCH, D), lambda i: (_oblk(i), 0)),
        pl.BlockSpec((2, RCH, HD), lambda i: (0, _oblk(i), 0)),
    ],
    out_shape=[jax.ShapeDtypeStruct((N, D), jnp.float32),
               jax.ShapeDtypeStruct((2, N, HD), jnp.float32)],
    scratch_shapes=[pltpu.VMEM((1, D), jnp.float32),
                    pltpu.VMEM((1, D), jnp.float32)],
)


def _tc_head_body(h_ref, batch_ref, gf_ref, w0_ref, b0_ref, w1_ref, b1_ref,
                  w2t_ref, b2_ref, out_ref):
    rows = lax.broadcasted_iota(jnp.int32, (B, N), 0)
    oh = (batch_ref[...] == rows).astype(jnp.float32)
    sums = jnp.dot(oh, h_ref[...], preferred_element_type=jnp.float32,
                   precision=_HI)
    cnt = jnp.sum(oh, axis=1, keepdims=True)
    pooled = sums / jnp.maximum(cnt, 1.0)
    z = jnp.dot(pooled, w0_ref[0:D, :], preferred_element_type=jnp.float32,
                precision=_HI)
    z = z + jnp.dot(gf_ref[...], w0_ref[D:, :],
                    preferred_element_type=jnp.float32, precision=_HI)
    z = jnp.maximum(z + b0_ref[...], 0.0)
    z = jnp.maximum(jnp.dot(z, w1_ref[...], preferred_element_type=jnp.float32,
                            precision=_HI) + b1_ref[...], 0.0)
    out_ref[...] = jnp.sum(z * w2t_ref[...], axis=1, keepdims=True) + b2_ref[...]


_tc_head = pl.pallas_call(
    _tc_head_body,
    out_shape=jax.ShapeDtypeStruct((B, 1), jnp.float32),
)


@jax.jit
def kernel(x, edge_index, batch, global_feats,
           W0, b0, g0, be0, W1, b1, g1, be1, W2, b2, g2, be2,
           mW0, mb0, mW1, mb1, mW2, mb2):
    src = edge_index[0]
    dst = edge_index[1]
    e = src.shape[0]
    cpt = -(-e // (NW * CHUNK))
    cpt = -(-cpt // NBUF) * NBUF
    e_pad = NW * cpt * CHUNK
    pad = e_pad - e
    pad_ar = jnp.arange(pad, dtype=jnp.int32)
    src_p = jnp.concatenate(
        [src, pad_ar % N]).reshape(NW, cpt, CHUNK)
    dst_p = jnp.concatenate(
        [dst, N + pad_ar % (N_ACC - N)]).reshape(NW, cpt, CHUNK)

    zeros_h = jnp.zeros((ZROWS, HD), jnp.float32)
    ones_h = jnp.ones((CHUNK, HD), jnp.float32)

    agg = _make_sc_aggregate(cpt)
    Ws = jnp.stack([W0, W0, W1, W2])
    bs = jnp.stack([b0, b0, b1, b2]).reshape(4, 1, -1)
    gs = jnp.stack([g0, g0, g1, g2]).reshape(4, 1, -1)
    bes = jnp.stack([be0, be0, be1, be2]).reshape(4, 1, -1)
    idx = jnp.arange(4, dtype=jnp.int32)
    modes = jnp.concatenate([jnp.ones((1, 16), jnp.int32),
                             jnp.zeros((3, 16), jnp.int32)])

    def step(carry, wgts):
        ui, hi, dinvi = carry
        i, mode, W, b, g, be = wgts
        accp = agg(mode, ui, src_p, dst_p, zeros_h, ones_h)

        def first(_):
            dinv, u0 = _tc_prep(accp, x)
            return (u0, hi, dinv)

        def rest(_):
            hn, un = _tc_layer(accp, ui, dinvi, W, b, g, be)
            return (un, hn, dinvi)

        return lax.cond(i == 0, first, rest, None), None

    carry0 = (jnp.ones((2, N, HD), jnp.float32),
              jnp.zeros((N, D), jnp.float32),
              jnp.zeros((N, 1), jnp.float32))
    (_, h, _), _ = lax.scan(step, carry0, (idx, modes, Ws, bs, gs, bes))

    out = _tc_head(h, batch.reshape(1, N), global_feats,
                   mW0, mb0.reshape(1, -1), mW1, mb1.reshape(1, -1),
                   mW2.reshape(1, -1), mb2.reshape(1, 1))
    return out.reshape(-1)


# speedup vs baseline: 19.7736x; 1.1187x over previous
import functools

import jax
import jax.numpy as jnp
from jax import lax
from jax.experimental import pallas as pl
from jax.experimental.pallas import tpu as pltpu
from jax.experimental.pallas import tpu_sc as plsc

N = 10000
D = 128
HD = D // 2
B = 64
NC = 2
NS = 16
NW = NC * NS
CHUNK = 128
N_ACC = 10112
ZROWS = N_ACC // NS

_HI = jax.lax.Precision.HIGHEST


@functools.cache
def _mesh():
    return plsc.VectorSubcoreMesh(core_axis_name="c", subcore_axis_name="s",
                                  num_cores=NC, num_subcores=NS)


NBUF = 4


def _sc_aggregate_body(mode_hbm, u_hbm, src_hbm, dst_hbm, zeros_hbm,
                       ones_hbm, out_hbm, mode_v, src_v, dst_v,
                       ones_v, bufs, acc, gsems, ssems):
    c = lax.axis_index("c")
    s = lax.axis_index("s")
    wid = s * NC + c
    cpt = dst_v.shape[0]
    pltpu.sync_copy(mode_hbm, mode_v)
    pltpu.sync_copy(src_hbm.at[wid], src_v)
    pltpu.sync_copy(dst_hbm.at[wid], dst_v)
    deg_mode = mode_v[...][0] == 1

    @pl.when(deg_mode)
    def _():
        pltpu.sync_copy(ones_hbm, ones_v)
        pltpu.sync_copy(zeros_hbm, acc.at[pl.ds(s * ZROWS, ZROWS)])
        plsc.subcore_barrier()

        def dbody(j, carry):
            pltpu.sync_copy(ones_v, acc.at[dst_v.at[j]], add=True)
            return carry

        lax.fori_loop(0, cpt, dbody, 0)
        plsc.subcore_barrier()
        pltpu.sync_copy(acc.at[pl.ds(s * ZROWS, ZROWS)],
                        out_hbm.at[c, pl.ds(s * ZROWS, ZROWS), pl.ds(0, HD)])

    @pl.when(jnp.logical_not(deg_mode))
    def _():
        for k in range(2):
            uk = u_hbm.at[k]
            pltpu.sync_copy(zeros_hbm, acc.at[pl.ds(s * ZROWS, ZROWS)])
            plsc.subcore_barrier()

            for b in range(NBUF - 1):
                pltpu.async_copy(uk.at[src_v.at[b]], bufs[b], gsems[b])

            def body(j, carry):
                jm = lax.rem(j, NBUF)
                for b in range(NBUF):
                    @pl.when(jm == b)
                    def _(b=b, uk=uk):
                        pltpu.make_async_copy(
                            uk.at[src_v.at[0]], bufs[b], gsems[b]).wait()
                        pltpu.async_copy(bufs[b], acc.at[dst_v.at[j]],
                                         ssems[b], add=True)

                        @pl.when(j + NBUF - 1 < cpt)
                        def _():
                            nb = (b + NBUF - 1) % NBUF

                            @pl.when(j >= 1)
                            def _():
                                pltpu.make_async_copy(
                                    bufs[nb], acc.at[dst_v.at[0]],
                                    ssems[nb]).wait()

                            pltpu.async_copy(uk.at[src_v.at[j + NBUF - 1]],
                                             bufs[nb], gsems[nb])
                return carry

            lax.fori_loop(0, cpt, body, 0)
            for b in range(NBUF):
                pltpu.make_async_copy(bufs[b], acc.at[dst_v.at[0]],
                                      ssems[b]).wait()
            plsc.subcore_barrier()
            pltpu.sync_copy(acc.at[pl.ds(s * ZROWS, ZROWS)],
                            out_hbm.at[c, pl.ds(s * ZROWS, ZROWS),
                                       pl.ds(k * HD, HD)])


def _make_sc_aggregate(cpt):
    return functools.partial(
        pl.kernel, _sc_aggregate_body,
        out_type=jax.ShapeDtypeStruct((NC, N_ACC, D), jnp.float32),
        mesh=_mesh(),
        scratch_types=[
            pltpu.VMEM((16,), jnp.int32),
            pltpu.VMEM((cpt, CHUNK), jnp.int32),
            pltpu.VMEM((cpt, CHUNK), jnp.int32),
            pltpu.VMEM((CHUNK, HD), jnp.float32),
            [pltpu.VMEM((CHUNK, HD), jnp.float32) for _ in range(NBUF)],
            pltpu.VMEM_SHARED((N_ACC, HD), jnp.float32),
            [pltpu.SemaphoreType.DMA for _ in range(NBUF)],
            [pltpu.SemaphoreType.DMA for _ in range(NBUF)],
        ],
        compiler_params=pltpu.CompilerParams(use_tc_tiling_on_sc=False),
    )()


RCH = 2000
NBLK = N // RCH


def _tc_prep_body(accp_ref, x_ref, dinv_ref, u_ref):
    deg = accp_ref[0, :, 0:1] + accp_ref[1, :, 0:1] + 1.0
    dinv = jnp.where(deg > 0, 1.0 / jnp.sqrt(deg), 0.0)
    dinv_ref[...] = dinv
    x = x_ref[...]
    u_ref[0] = x[:, 0:HD] * dinv
    u_ref[1] = x[:, HD:D] * dinv


_tc_prep = pl.pallas_call(
    _tc_prep_body,
    grid=(NBLK,),
    in_specs=[
        pl.BlockSpec((2, RCH, D), lambda i: (0, i, 0)),
        pl.BlockSpec((RCH, D), lambda i: (i, 0)),
    ],
    out_specs=[
        pl.BlockSpec((RCH, 1), lambda i: (i, 0)),
        pl.BlockSpec((2, RCH, HD), lambda i: (0, i, 0)),
    ],
    out_shape=[jax.ShapeDtypeStruct((N, 1), jnp.float32),
               jax.ShapeDtypeStruct((2, N, HD), jnp.float32)],
)


def _zblock(accp_ref, u_ref, dinv_ref, w_ref, b_ref):
    dinv = dinv_ref[...]
    acs = accp_ref[0] + accp_ref[1]
    t0 = (acs[:, 0:HD] + u_ref[0]) * dinv
    t1 = (acs[:, HD:D] + u_ref[1]) * dinv
    z = jnp.dot(t0, w_ref[0:HD, :], preferred_element_type=jnp.float32,
                precision=_HI)
    z = z + jnp.dot(t1, w_ref[HD:D, :], preferred_element_type=jnp.float32,
                    precision=_HI)
    return z + b_ref[...]


def _tc_layer_body(accp_ref, u_ref, dinv_ref, w_ref, b_ref, g_ref, be_ref,
                   h_ref, un_ref, s1_ref, s2_ref):
    i = pl.program_id(0)

    @pl.when(i == 0)
    def _():
        s1_ref[...] = jnp.zeros_like(s1_ref)
        s2_ref[...] = jnp.zeros_like(s2_ref)

    z = _zblock(accp_ref, u_ref, dinv_ref, w_ref, b_ref)

    @pl.when(i < NBLK)
    def _():
        s1_ref[...] += jnp.sum(z, axis=0, keepdims=True)
        s2_ref[...] += jnp.sum(z * z, axis=0, keepdims=True)

    @pl.when(i >= NBLK)
    def _():
        m = s1_ref[...] / N
        var = s2_ref[...] / N - m * m
        rstd = 1.0 / jnp.sqrt(var + 1e-5)
        h = jnp.maximum((z - m) * rstd * g_ref[...] + be_ref[...], 0.0)
        h_ref[...] = h
        hd = h * dinv_ref[...]
        un_ref[0] = hd[:, 0:HD]
        un_ref[1] = hd[:, HD:D]


def _blk(i):
    return lax.rem(i, NBLK)


def _oblk(i):
    return jnp.maximum(i - NBLK, 0)


_tc_layer = pl.pallas_call(
    _tc_layer_body,
    grid=(2 * NBLK,),
    in_specs=[
        pl.BlockSpec((2, RCH, D), lambda i: (0, _blk(i), 0)),
        pl.BlockSpec((2, RCH, HD), lambda i: (0, _blk(i), 0)),
        pl.BlockSpec((RCH, 1), lambda i: (_blk(i), 0)),
        pl.BlockSpec((D, D), lambda i: (0, 0)),
        pl.BlockSpec((1, D), lambda i: (0, 0)),
        pl.BlockSpec((1, D), lambda i: (0, 0)),
        pl.BlockSpec((1, D), lambda i: (0, 0)),
    ],
    out_specs=[
        pl.BlockSpec((RCH, D), lambda i: (_oblk(i), 0)),
        pl.BlockSpec((2, RCH, HD), lambda i: (0, _oblk(i), 0)),
    ],
    out_shape=[jax.ShapeDtypeStruct((N, D), jnp.float32),
               jax.ShapeDtypeStruct((2, N, HD), jnp.float32)],
    scratch_shapes=[pltpu.VMEM((1, D), jnp.float32),
                    pltpu.VMEM((1, D), jnp.float32)],
)


def _tc_head_body(h_ref, batch_ref, gf_ref, w0_ref, b0_ref, w1_ref, b1_ref,
                  w2t_ref, b2_ref, out_ref):
    rows = lax.broadcasted_iota(jnp.int32, (B, N), 0)
    oh = (batch_ref[...] == rows).astype(jnp.float32)
    sums = jnp.dot(oh, h_ref[...], preferred_element_type=jnp.float32,
                   precision=_HI)
    cnt = jnp.sum(oh, axis=1, keepdims=True)
    pooled = sums / jnp.maximum(cnt, 1.0)
    z = jnp.dot(pooled, w0_ref[0:D, :], preferred_element_type=jnp.float32,
                precision=_HI)
    z = z + jnp.dot(gf_ref[...], w0_ref[D:, :],
                    preferred_element_type=jnp.float32, precision=_HI)
    z = jnp.maximum(z + b0_ref[...], 0.0)
    z = jnp.maximum(jnp.dot(z, w1_ref[...], preferred_element_type=jnp.float32,
                            precision=_HI) + b1_ref[...], 0.0)
    out_ref[...] = jnp.sum(z * w2t_ref[...], axis=1, keepdims=True) + b2_ref[...]


_tc_head = pl.pallas_call(
    _tc_head_body,
    out_shape=jax.ShapeDtypeStruct((B, 1), jnp.float32),
)


@jax.jit
def kernel(x, edge_index, batch, global_feats,
           W0, b0, g0, be0, W1, b1, g1, be1, W2, b2, g2, be2,
           mW0, mb0, mW1, mb1, mW2, mb2):
    src = edge_index[0]
    dst = edge_index[1]
    e = src.shape[0]
    cpt = -(-e // (NW * CHUNK))
    cpt = -(-cpt // NBUF) * NBUF
    e_pad = NW * cpt * CHUNK
    pad = e_pad - e
    pad_ar = jnp.arange(pad, dtype=jnp.int32)
    src_p = jnp.concatenate([src, pad_ar % N]).reshape(NW, cpt, CHUNK)
    dst_p = jnp.concatenate(
        [dst, N + pad_ar % (N_ACC - N)]).reshape(NW, cpt, CHUNK)

    zeros_h = jnp.zeros((ZROWS, HD), jnp.float32)
    ones_h = jnp.ones((CHUNK, HD), jnp.float32)

    agg = _make_sc_aggregate(cpt)
    Ws = jnp.stack([W0, W0, W1, W2])
    bs = jnp.stack([b0, b0, b1, b2]).reshape(4, 1, -1)
    gs = jnp.stack([g0, g0, g1, g2]).reshape(4, 1, -1)
    bes = jnp.stack([be0, be0, be1, be2]).reshape(4, 1, -1)
    idx = jnp.arange(4, dtype=jnp.int32)
    modes = jnp.concatenate([jnp.ones((1, 16), jnp.int32),
                             jnp.zeros((3, 16), jnp.int32)])

    def step(carry, wgts):
        ui, hi, dinvi = carry
        i, mode, W, b, g, be = wgts
        accp = agg(mode, ui, src_p, dst_p, zeros_h, ones_h)

        def first(_):
            dinv, u0 = _tc_prep(accp, x)
            return (u0, hi, dinv)

        def rest(_):
            hn, un = _tc_layer(accp, ui, dinvi, W, b, g, be)
            return (un, hn, dinvi)

        return lax.cond(i == 0, first, rest, None), None

    carry0 = (jnp.ones((2, N, HD), jnp.float32),
              jnp.zeros((N, D), jnp.float32),
              jnp.zeros((N, 1), jnp.float32))
    (_, h, _), _ = lax.scan(step, carry0, (idx, modes, Ws, bs, gs, bes))

    out = _tc_head(h, batch.reshape(1, N), global_feats,
                   mW0, mb0.reshape(1, -1), mW1, mb1.reshape(1, -1),
                   mW2.reshape(1, -1), mb2.reshape(1, 1))
    return out.reshape(-1)


# speedup vs baseline: 20.8604x; 1.0550x over previous
import functools

import jax
import jax.numpy as jnp
from jax import lax
from jax.experimental import pallas as pl
from jax.experimental.pallas import tpu as pltpu
from jax.experimental.pallas import tpu_sc as plsc

N = 10000
D = 128
HD = D // 2
B = 64
NC = 2
NS = 16
NW = NC * NS
CHUNK = 128
N_ACC = 10112
ZROWS = N_ACC // NS

_HI = jax.lax.Precision.HIGHEST


@functools.cache
def _mesh():
    return plsc.VectorSubcoreMesh(core_axis_name="c", subcore_axis_name="s",
                                  num_cores=NC, num_subcores=NS)


NBUF = 4


def _sc_aggregate_body(mode_hbm, u_hbm, src_hbm, dst_hbm, zeros_hbm,
                       ones_hbm, out_hbm, mode_v, src_v, dst_v,
                       ones_v, bufs, acc, gsems, ssems):
    c = lax.axis_index("c")
    s = lax.axis_index("s")
    wid = s * NC + c
    cpt = dst_v.shape[0]
    pltpu.sync_copy(mode_hbm, mode_v)
    pltpu.sync_copy(src_hbm.at[wid], src_v)
    pltpu.sync_copy(dst_hbm.at[wid], dst_v)
    deg_mode = mode_v[...][0] == 1

    @pl.when(deg_mode)
    def _():
        pltpu.sync_copy(ones_hbm, ones_v)
        pltpu.sync_copy(zeros_hbm, acc.at[pl.ds(s * ZROWS, ZROWS)])
        plsc.subcore_barrier()

        def dbody(j, carry):
            pltpu.sync_copy(ones_v, acc.at[dst_v.at[j]], add=True)
            return carry

        lax.fori_loop(0, cpt, dbody, 0)
        plsc.subcore_barrier()
        pltpu.sync_copy(acc.at[pl.ds(s * ZROWS, ZROWS)],
                        out_hbm.at[c, pl.ds(s * ZROWS, ZROWS), pl.ds(0, HD)])

    @pl.when(jnp.logical_not(deg_mode))
    def _():
        for k in range(2):
            uk = u_hbm.at[k]
            pltpu.sync_copy(zeros_hbm, acc.at[pl.ds(s * ZROWS, ZROWS)])
            plsc.subcore_barrier()

            for b in range(NBUF - 1):
                pltpu.async_copy(uk.at[src_v.at[b]], bufs[b], gsems[b])

            def body(j, carry):
                jm = lax.rem(j, NBUF)
                for b in range(NBUF):
                    @pl.when(jm == b)
                    def _(b=b, uk=uk):
                        pltpu.make_async_copy(
                            uk.at[src_v.at[0]], bufs[b], gsems[b]).wait()
                        pltpu.async_copy(bufs[b], acc.at[dst_v.at[j]],
                                         ssems[b], add=True)

                        @pl.when(j + NBUF - 1 < cpt)
                        def _():
                            nb = (b + NBUF - 1) % NBUF

                            @pl.when(j >= 1)
                            def _():
                                pltpu.make_async_copy(
                                    bufs[nb], acc.at[dst_v.at[0]],
                                    ssems[nb]).wait()

                            pltpu.async_copy(uk.at[src_v.at[j + NBUF - 1]],
                                             bufs[nb], gsems[nb])
                return carry

            lax.fori_loop(0, cpt, body, 0)
            for b in range(NBUF):
                pltpu.make_async_copy(bufs[b], acc.at[dst_v.at[0]],
                                      ssems[b]).wait()
            plsc.subcore_barrier()
            pltpu.sync_copy(acc.at[pl.ds(s * ZROWS, ZROWS)],
                            out_hbm.at[c, pl.ds(s * ZROWS, ZROWS),
                                       pl.ds(k * HD, HD)])


def _make_sc_aggregate(cpt):
    return functools.partial(
        pl.kernel, _sc_aggregate_body,
        out_type=jax.ShapeDtypeStruct((NC, N_ACC, D), jnp.float32),
        mesh=_mesh(),
        scratch_types=[
            pltpu.VMEM((16,), jnp.int32),
            pltpu.VMEM((cpt, CHUNK), jnp.int32),
            pltpu.VMEM((cpt, CHUNK), jnp.int32),
            pltpu.VMEM((CHUNK, HD), jnp.float32),
            [pltpu.VMEM((CHUNK, HD), jnp.float32) for _ in range(NBUF)],
            pltpu.VMEM_SHARED((N_ACC, HD), jnp.float32),
            [pltpu.SemaphoreType.DMA for _ in range(NBUF)],
            [pltpu.SemaphoreType.DMA for _ in range(NBUF)],
        ],
        compiler_params=pltpu.CompilerParams(use_tc_tiling_on_sc=False),
    )()


RCH = 2000
NBLK = N // RCH


def _tc_prep_body(accp_ref, x_ref, w_ref, dinv_ref, u_ref):
    deg = accp_ref[0, :, 0:1] + accp_ref[1, :, 0:1] + 1.0
    dinv = jnp.where(deg > 0, 1.0 / jnp.sqrt(deg), 0.0)
    dinv_ref[...] = dinv
    xw = jnp.dot(x_ref[...], w_ref[...], preferred_element_type=jnp.float32)
    u = xw * dinv
    u_ref[0] = u[:, 0:HD]
    u_ref[1] = u[:, HD:D]


_tc_prep = pl.pallas_call(
    _tc_prep_body,
    grid=(NBLK,),
    in_specs=[
        pl.BlockSpec((2, RCH, D), lambda i: (0, i, 0)),
        pl.BlockSpec((RCH, D), lambda i: (i, 0)),
        pl.BlockSpec((D, D), lambda i: (0, 0)),
    ],
    out_specs=[
        pl.BlockSpec((RCH, 1), lambda i: (i, 0)),
        pl.BlockSpec((2, RCH, HD), lambda i: (0, i, 0)),
    ],
    out_shape=[jax.ShapeDtypeStruct((N, 1), jnp.float32),
               jax.ShapeDtypeStruct((2, N, HD), jnp.float32)],
)


def _zblock(accp_ref, u_ref, dinv_ref, b_ref):
    dinv = dinv_ref[...]
    acs = accp_ref[0] + accp_ref[1]
    u = jnp.concatenate([u_ref[0], u_ref[1]], axis=1)
    return (acs + u) * dinv + b_ref[...]


def _tc_layer_body(accp_ref, u_ref, dinv_ref, wn_ref, b_ref, g_ref, be_ref,
                   h_ref, un_ref, s1_ref, s2_ref):
    i = pl.program_id(0)

    @pl.when(i == 0)
    def _():
        s1_ref[...] = jnp.zeros_like(s1_ref)
        s2_ref[...] = jnp.zeros_like(s2_ref)

    z = _zblock(accp_ref, u_ref, dinv_ref, b_ref)

    @pl.when(i < NBLK)
    def _():
        mb = jnp.mean(z, axis=0, keepdims=True)
        zb = z - mb
        m2b = jnp.sum(zb * zb, axis=0, keepdims=True)
        n_old = (i * RCH).astype(jnp.float32)
        n_new = n_old + RCH
        delta = mb - s1_ref[...]
        s1_ref[...] += delta * (RCH / n_new)
        s2_ref[...] += m2b + delta * delta * (n_old * RCH / n_new)

    @pl.when(i >= NBLK)
    def _():
        m = s1_ref[...]
        var = s2_ref[...] / N
        rstd = 1.0 / jnp.sqrt(var + 1e-5)
        h = jnp.maximum((z - m) * rstd * g_ref[...] + be_ref[...], 0.0)
        h_ref[...] = h
        hw = jnp.dot(h, wn_ref[...], preferred_element_type=jnp.float32)
        un = hw * dinv_ref[...]
        un_ref[0] = un[:, 0:HD]
        un_ref[1] = un[:, HD:D]


def _blk(i):
    return lax.rem(i, NBLK)


def _oblk(i):
    return jnp.maximum(i - NBLK, 0)


_tc_layer = pl.pallas_call(
    _tc_layer_body,
    grid=(2 * NBLK,),
    in_specs=[
        pl.BlockSpec((2, RCH, D), lambda i: (0, _blk(i), 0)),
        pl.BlockSpec((2, RCH, HD), lambda i: (0, _blk(i), 0)),
        pl.BlockSpec((RCH, 1), lambda i: (_blk(i), 0)),
        pl.BlockSpec((D, D), lambda i: (0, 0)),
        pl.BlockSpec((1, D), lambda i: (0, 0)),
        pl.BlockSpec((1, D), lambda i: (0, 0)),
        pl.BlockSpec((1, D), lambda i: (0, 0)),
    ],
    out_specs=[
        pl.BlockSpec((RCH, D), lambda i: (_oblk(i), 0)),
        pl.BlockSpec((2, RCH, HD), lambda i: (0, _oblk(i), 0)),
    ],
    out_shape=[jax.ShapeDtypeStruct((N, D), jnp.float32),
               jax.ShapeDtypeStruct((2, N, HD), jnp.float32)],
    scratch_shapes=[pltpu.VMEM((1, D), jnp.float32),
                    pltpu.VMEM((1, D), jnp.float32)],
)


def _tc_head_body(h_ref, batch_ref, gf_ref, w0_ref, b0_ref, w1_ref, b1_ref,
                  w2t_ref, b2_ref, out_ref):
    rows = lax.broadcasted_iota(jnp.int32, (B, N), 0)
    oh = (batch_ref[...] == rows).astype(jnp.float32)
    sums = jnp.dot(oh, h_ref[...], preferred_element_type=jnp.float32,
                   precision=_HI)
    cnt = jnp.sum(oh, axis=1, keepdims=True)
    pooled = sums / jnp.maximum(cnt, 1.0)
    z = jnp.dot(pooled, w0_ref[0:D, :], preferred_element_type=jnp.float32)
    z = z + jnp.dot(gf_ref[...], w0_ref[D:, :],
                    preferred_element_type=jnp.float32)
    z = jnp.maximum(z + b0_ref[...], 0.0)
    z = jnp.maximum(jnp.dot(z, w1_ref[...],
                            preferred_element_type=jnp.float32)
                    + b1_ref[...], 0.0)
    out_ref[...] = jnp.sum(z * w2t_ref[...], axis=1, keepdims=True) + b2_ref[...]


_tc_head = pl.pallas_call(
    _tc_head_body,
    out_shape=jax.ShapeDtypeStruct((B, 1), jnp.float32),
)


@jax.jit
def kernel(x, edge_index, batch, global_feats,
           W0, b0, g0, be0, W1, b1, g1, be1, W2, b2, g2, be2,
           mW0, mb0, mW1, mb1, mW2, mb2):
    src = edge_index[0]
    dst = edge_index[1]
    e = src.shape[0]
    cpt = -(-e // (NW * CHUNK))
    cpt = -(-cpt // NBUF) * NBUF
    e_pad = NW * cpt * CHUNK
    pad = e_pad - e
    pad_ar = jnp.arange(pad, dtype=jnp.int32)
    src_p = jnp.concatenate([src, pad_ar % N]).reshape(NW, cpt, CHUNK)
    dst_p = jnp.concatenate(
        [dst, N + pad_ar % (N_ACC - N)]).reshape(NW, cpt, CHUNK)

    zeros_h = jnp.zeros((ZROWS, HD), jnp.float32)
    ones_h = jnp.ones((CHUNK, HD), jnp.float32)

    agg = _make_sc_aggregate(cpt)
    Ws = jnp.stack([W0, W1, W2, W2])
    bs = jnp.stack([b0, b0, b1, b2]).reshape(4, 1, -1)
    gs = jnp.stack([g0, g0, g1, g2]).reshape(4, 1, -1)
    bes = jnp.stack([be0, be0, be1, be2]).reshape(4, 1, -1)
    idx = jnp.arange(4, dtype=jnp.int32)
    modes = jnp.concatenate([jnp.ones((1, 16), jnp.int32),
                             jnp.zeros((3, 16), jnp.int32)])

    def step(carry, wgts):
        ui, hi, dinvi = carry
        i, mode, W, b, g, be = wgts
        accp = agg(mode, ui, src_p, dst_p, zeros_h, ones_h)

        def first(_):
            dinv, u0 = _tc_prep(accp, x, W)
            return (u0, hi, dinv)

        def rest(_):
            hn, un = _tc_layer(accp, ui, dinvi, W, b, g, be)
            return (un, hn, dinvi)

        return lax.cond(i == 0, first, rest, None), None

    carry0 = (jnp.ones((2, N, HD), jnp.float32),
              jnp.zeros((N, D), jnp.float32),
              jnp.zeros((N, 1), jnp.float32))
    (_, h, _), _ = lax.scan(step, carry0, (idx, modes, Ws, bs, gs, bes))

    out = _tc_head(h, batch.reshape(1, N), global_feats,
                   mW0, mb0.reshape(1, -1), mW1, mb1.reshape(1, -1),
                   mW2.reshape(1, -1), mb2.reshape(1, 1))
    return out.reshape(-1)
